# Initial kernel scaffold; baseline (speedup 1.0000x reference)
import jax
import jax.numpy as jnp
from jax.experimental import pallas as pl


def kernel(x, edge_index, edge_attr, W1, b1, W2, b2):
    raise NotImplementedError("write your pallas kernel here")


# speedup vs baseline: 3.6878x; 3.6878x over previous
import functools

import jax
import jax.numpy as jnp
from jax import lax
from jax.experimental import pallas as pl
from jax.experimental.pallas import tpu as pltpu
from jax.experimental.pallas import tpu_sc as plsc

N = 10000
E = 320000
DF = 128
DE = 16
OC = 32

NC = 2
NS = 16
NW = NC * NS
L = 16

E_PER_W = E // NW
G2 = 40
C2 = 200
NG2 = C2 // G2
K2C = E_PER_W // C2

CJ = 1600
K4_CHUNKS = E // CJ

B3 = 6400
G3 = E // B3


def _k1_body(x_ref, w_ref, pd_ref, ps_ref):
    p = jnp.dot(x_ref[...], w_ref[...], preferred_element_type=jnp.float32)
    pd_ref[...] = p[:, :OC]
    ps_ref[...] = p[:, OC:]


def _k1(x, w_ds):
    return pl.pallas_call(
        _k1_body,
        out_shape=(
            jax.ShapeDtypeStruct((N, OC), jnp.float32),
            jax.ShapeDtypeStruct((N, OC), jnp.float32),
        ),
    )(x, w_ds)


def _k2_body(pd_hbm, ps_hbm, src_hbm, dst_hbm, out_hbm,
             idxs0, idxs1, idxd0, idxd1, rd0, rd1, rs0, rs1,
             semi0, semi1, semg0, semg1):
    wid = lax.axis_index("s") * NC + lax.axis_index("c")
    ebase = wid * E_PER_W
    idxs = (idxs0, idxs1)
    idxd = (idxd0, idxd1)
    rd = (rd0, rd1)
    rs = (rs0, rs1)
    semi = (semi0, semi1)
    semg = (semg0, semg1)

    def issue_idx(j, b):
        base = ebase + j * C2
        pltpu.async_copy(src_hbm.at[pl.ds(base, C2)], idxs[b], semi[b])
        pltpu.async_copy(dst_hbm.at[pl.ds(base, C2)], idxd[b], semi[b])

    def wait_idx(b):
        pltpu.make_async_copy(src_hbm.at[pl.ds(0, C2)], idxs[b], semi[b]).wait()
        pltpu.make_async_copy(dst_hbm.at[pl.ds(0, C2)], idxd[b], semi[b]).wait()

    def issue_gathers(b):
        for g in range(NG2):
            sl = pl.ds(g * G2, G2)
            pltpu.async_copy(pd_hbm.at[idxd[b].at[sl]], rd[b].at[sl], semg[b])
            pltpu.async_copy(ps_hbm.at[idxs[b].at[sl]], rs[b].at[sl], semg[b])

    def wait_gathers(b):
        pltpu.make_async_copy(pd_hbm.at[pl.ds(0, C2)], rd[b], semg[b]).wait()
        pltpu.make_async_copy(ps_hbm.at[pl.ds(0, C2)], rs[b], semg[b]).wait()

    def process(jj, b):
        wait_gathers(b)

        @pl.when(jj + 1 < K2C)
        def _advance():
            wait_idx(1 - b)
            issue_gathers(1 - b)

        @pl.when(jj + 2 < K2C)
        def _prefetch():
            issue_idx(jj + 2, b)

        def row(r, c):
            rd[b][r, pl.ds(0, L)] = rd[b][r, pl.ds(0, L)] + rs[b][r, pl.ds(0, L)]
            rd[b][r, pl.ds(L, L)] = rd[b][r, pl.ds(L, L)] + rs[b][r, pl.ds(L, L)]
            return c

        lax.fori_loop(0, C2, row, 0)
        pltpu.sync_copy(rd[b], out_hbm.at[pl.ds(ebase + jj * C2, C2)])

    issue_idx(0, 0)
    wait_idx(0)
    issue_gathers(0)
    issue_idx(1, 1)

    def pair(jp, c):
        process(2 * jp, 0)
        process(2 * jp + 1, 1)
        return c

    lax.fori_loop(0, K2C // 2, pair, 0)


@functools.partial(
    pl.kernel,
    mesh=plsc.VectorSubcoreMesh(core_axis_name="c", subcore_axis_name="s"),
    out_type=jax.ShapeDtypeStruct((E, OC), jnp.float32),
    scratch_types=[
        pltpu.VMEM((C2,), jnp.int32),
        pltpu.VMEM((C2,), jnp.int32),
        pltpu.VMEM((C2,), jnp.int32),
        pltpu.VMEM((C2,), jnp.int32),
        pltpu.VMEM((C2, OC), jnp.float32),
        pltpu.VMEM((C2, OC), jnp.float32),
        pltpu.VMEM((C2, OC), jnp.float32),
        pltpu.VMEM((C2, OC), jnp.float32),
        pltpu.SemaphoreType.DMA,
        pltpu.SemaphoreType.DMA,
        pltpu.SemaphoreType.DMA,
        pltpu.SemaphoreType.DMA,
    ],
    compiler_params=pltpu.CompilerParams(use_tc_tiling_on_sc=False, needs_layout_passes=False),
)
def _k2(pd_hbm, ps_hbm, src_hbm, dst_hbm, out_hbm,
        idxs0, idxs1, idxd0, idxd1, rd0, rd1, rs0, rs1,
        semi0, semi1, semg0, semg1):
    _k2_body(pd_hbm, ps_hbm, src_hbm, dst_hbm, out_hbm,
             idxs0, idxs1, idxd0, idxd1, rd0, rd1, rs0, rs1,
             semi0, semi1, semg0, semg1)


def _k3_body(zp_ref, ea_ref, w1e_ref, b1_ref, w2t_ref, ht_ref):
    pe = jnp.dot(ea_ref[...], w1e_ref[...], preferred_element_type=jnp.float32)
    a = jnp.maximum(zp_ref[...] + pe + b1_ref[...], 0.0)
    ht_ref[...] = lax.dot_general(
        w2t_ref[...], a, (((1,), (1,)), ((), ())),
        preferred_element_type=jnp.float32)


def _k3(zp, ea, w1e, b1_2d, w2t):
    return pl.pallas_call(
        _k3_body,
        grid=(G3,),
        in_specs=[
            pl.BlockSpec((B3, OC), lambda i: (i, 0)),
            pl.BlockSpec((B3, DE), lambda i: (i, 0)),
            pl.BlockSpec((DE, OC), lambda i: (0, 0)),
            pl.BlockSpec((1, OC), lambda i: (0, 0)),
            pl.BlockSpec((OC, OC), lambda i: (0, 0)),
        ],
        out_specs=pl.BlockSpec((OC, B3), lambda i: (0, i)),
        out_shape=jax.ShapeDtypeStruct((OC, E), jnp.float32),
    )(zp, ea, w1e, b1_2d, w2t)


SPREAD = 8
ACC8 = SPREAD * N


def _k4_body(ht_hbm, dst_hbm, out_hbm, idx0, idx1, val0, val1, acc_v, sem0, sem1):
    wid = lax.axis_index("s") * NC + lax.axis_index("c")
    idxb = (idx0, idx1)
    valb = (val0, val1)
    sems = (sem0, sem1)
    lanes = lax.iota(jnp.int32, L)
    lane_off = (lanes & (SPREAD - 1)) * N
    perm = lanes ^ 8
    neg_inf = jnp.full((L,), -jnp.inf, dtype=jnp.float32)
    zeros = jnp.zeros((L,), jnp.float32)

    def ini(i, c):
        acc_v[pl.ds(i * L, L)] = neg_inf
        return c

    lax.fori_loop(0, ACC8 // L, ini, 0)

    def issue(j, b):
        pltpu.async_copy(dst_hbm.at[pl.ds(j * CJ, CJ)], idxb[b], sems[b])
        pltpu.async_copy(ht_hbm.at[wid, pl.ds(j * CJ, CJ)], valb[b], sems[b])

    def waitc(b):
        pltpu.make_async_copy(dst_hbm.at[pl.ds(0, CJ)], idxb[b], sems[b]).wait()
        pltpu.make_async_copy(ht_hbm.at[wid, pl.ds(0, CJ)], valb[b], sems[b]).wait()

    def process(jj, b):
        @pl.when(jj + 1 < K4_CHUNKS)
        def _prefetch():
            issue(jj + 1, 1 - b)

        waitc(b)
        idx_v = idxb[b]
        val_v = valb[b]

        def vec1(v, flag):
            idx = idx_v[pl.ds(v * L, L)]
            val = val_v[pl.ds(v * L, L)]
            idx8 = idx + lane_off
            rot = idx.at[perm].get(mode="promise_in_bounds")
            flag = jnp.maximum(flag, jnp.where(idx == rot, 1.0, 0.0))
            cur = plsc.load_gather(acc_v, [idx8])
            plsc.store_scatter(acc_v, [idx8], jnp.maximum(cur, val))
            return flag

        flag = lax.fori_loop(0, CJ // L, vec1, zeros)

        @pl.when(jnp.max(flag) > 0.0)
        def _fix():
            def wbody(go):
                def vec2(v, bad):
                    idx = idx_v[pl.ds(v * L, L)]
                    val = val_v[pl.ds(v * L, L)]
                    idx8 = idx + lane_off
                    chk = plsc.load_gather(acc_v, [idx8])
                    m = val > chk
                    plsc.store_scatter(acc_v, [idx8], val, mask=m)
                    return jnp.maximum(bad, jnp.where(m, 1.0, 0.0))

                bad = lax.fori_loop(0, CJ // L, vec2, zeros)
                return jnp.max(bad) > 0.0

            lax.while_loop(lambda go: go, wbody, jnp.bool_(True))

    issue(0, 0)

    def pair(jp, c):
        process(2 * jp, 0)
        process(2 * jp + 1, 1)
        return c

    lax.fori_loop(0, K4_CHUNKS // 2, pair, 0)

    def merge(i, c):
        b0 = acc_v[pl.ds(i * L, L)]
        for b in range(1, SPREAD):
            b0 = jnp.maximum(b0, acc_v[pl.ds(b * N + i * L, L)])
        acc_v[pl.ds(i * L, L)] = b0
        return c

    lax.fori_loop(0, N // L, merge, 0)
    pltpu.sync_copy(acc_v.at[pl.ds(0, N)], out_hbm.at[wid])


@functools.partial(
    pl.kernel,
    mesh=plsc.VectorSubcoreMesh(core_axis_name="c", subcore_axis_name="s"),
    out_type=jax.ShapeDtypeStruct((OC, N), jnp.float32),
    scratch_types=[
        pltpu.VMEM((CJ,), jnp.int32),
        pltpu.VMEM((CJ,), jnp.int32),
        pltpu.VMEM((CJ,), jnp.float32),
        pltpu.VMEM((CJ,), jnp.float32),
        pltpu.VMEM((ACC8,), jnp.float32),
        pltpu.SemaphoreType.DMA,
        pltpu.SemaphoreType.DMA,
    ],
    compiler_params=pltpu.CompilerParams(use_tc_tiling_on_sc=False, needs_layout_passes=False),
)
def _k4(ht_hbm, dst_hbm, out_hbm, idx0, idx1, val0, val1, acc_v, sem0, sem1):
    _k4_body(ht_hbm, dst_hbm, out_hbm, idx0, idx1, val0, val1, acc_v, sem0, sem1)


def kernel(x, edge_index, edge_attr, W1, b1, W2, b2):
    src = edge_index[0]
    dst = edge_index[1]
    w_ds = jnp.concatenate([W1[:DF], W1[DF:2 * DF]], axis=1)
    w1e = W1[2 * DF:]

    pd, ps = _k1(x, w_ds)
    zp = _k2(pd, ps, src, dst)
    ht = _k3(zp, edge_attr, w1e, b1.reshape(1, OC), W2.T)
    acc = _k4(ht, dst).T
    return jnp.where(jnp.isfinite(acc), acc + b2[None, :], 0.0)


# speedup vs baseline: 4.1546x; 1.1266x over previous
import functools

import jax
import jax.numpy as jnp
from jax import lax
from jax.experimental import pallas as pl
from jax.experimental.pallas import tpu as pltpu
from jax.experimental.pallas import tpu_sc as plsc

N = 10000
E = 320000
DF = 128
DE = 16
OC = 32

NC = 2
NS = 16
NW = NC * NS
L = 16

E_PER_W = E // NW
G2 = 40
C2 = 200
NG2 = C2 // G2
K2C = E_PER_W // C2

PAIRS = OC // 2
E2 = E // 2
CJ = 1600
K4C = E2 // CJ

B3 = 6400
G3 = E // B3


def _k1_body(x_ref, w_ref, pd_ref, ps_ref):
    p = jnp.dot(x_ref[...], w_ref[...], preferred_element_type=jnp.float32)
    pd_ref[...] = p[:, :OC]
    ps_ref[...] = p[:, OC:]


def _k1(x, w_ds):
    return pl.pallas_call(
        _k1_body,
        out_shape=(
            jax.ShapeDtypeStruct((N, OC), jnp.float32),
            jax.ShapeDtypeStruct((N, OC), jnp.float32),
        ),
    )(x, w_ds)


def _k2_body(pd_hbm, ps_hbm, src_hbm, dst_hbm, out_hbm,
             idxs0, idxs1, idxd0, idxd1, rd0, rd1, rs0, rs1,
             semi0, semi1, semg0, semg1):
    wid = lax.axis_index("s") * NC + lax.axis_index("c")
    ebase = wid * E_PER_W
    idxs = (idxs0, idxs1)
    idxd = (idxd0, idxd1)
    rd = (rd0, rd1)
    rs = (rs0, rs1)
    semi = (semi0, semi1)
    semg = (semg0, semg1)

    def issue_idx(j, b):
        base = ebase + j * C2
        pltpu.async_copy(src_hbm.at[pl.ds(base, C2)], idxs[b], semi[b])
        pltpu.async_copy(dst_hbm.at[pl.ds(base, C2)], idxd[b], semi[b])

    def wait_idx(b):
        pltpu.make_async_copy(src_hbm.at[pl.ds(0, C2)], idxs[b], semi[b]).wait()
        pltpu.make_async_copy(dst_hbm.at[pl.ds(0, C2)], idxd[b], semi[b]).wait()

    def issue_gathers(b):
        for g in range(NG2):
            sl = pl.ds(g * G2, G2)
            pltpu.async_copy(pd_hbm.at[idxd[b].at[sl]], rd[b].at[sl], semg[b])
            pltpu.async_copy(ps_hbm.at[idxs[b].at[sl]], rs[b].at[sl], semg[b])

    def wait_gathers(b):
        pltpu.make_async_copy(pd_hbm.at[pl.ds(0, C2)], rd[b], semg[b]).wait()
        pltpu.make_async_copy(ps_hbm.at[pl.ds(0, C2)], rs[b], semg[b]).wait()

    def process(jj, b):
        wait_gathers(b)

        @pl.when(jj + 1 < K2C)
        def _advance():
            wait_idx(1 - b)
            issue_gathers(1 - b)

        @pl.when(jj + 2 < K2C)
        def _prefetch():
            issue_idx(jj + 2, b)

        def row(r, c):
            rd[b][r, pl.ds(0, L)] = rd[b][r, pl.ds(0, L)] + rs[b][r, pl.ds(0, L)]
            rd[b][r, pl.ds(L, L)] = rd[b][r, pl.ds(L, L)] + rs[b][r, pl.ds(L, L)]
            return c

        lax.fori_loop(0, C2, row, 0)
        pltpu.sync_copy(rd[b], out_hbm.at[pl.ds(ebase + jj * C2, C2)])

    issue_idx(0, 0)
    wait_idx(0)
    issue_gathers(0)
    issue_idx(1, 1)

    def pair(jp, c):
        process(2 * jp, 0)
        process(2 * jp + 1, 1)
        return c

    lax.fori_loop(0, K2C // 2, pair, 0)


@functools.partial(
    pl.kernel,
    mesh=plsc.VectorSubcoreMesh(core_axis_name="c", subcore_axis_name="s"),
    out_type=jax.ShapeDtypeStruct((E, OC), jnp.float32),
    scratch_types=[
        pltpu.VMEM((C2,), jnp.int32),
        pltpu.VMEM((C2,), jnp.int32),
        pltpu.VMEM((C2,), jnp.int32),
        pltpu.VMEM((C2,), jnp.int32),
        pltpu.VMEM((C2, OC), jnp.float32),
        pltpu.VMEM((C2, OC), jnp.float32),
        pltpu.VMEM((C2, OC), jnp.float32),
        pltpu.VMEM((C2, OC), jnp.float32),
        pltpu.SemaphoreType.DMA,
        pltpu.SemaphoreType.DMA,
        pltpu.SemaphoreType.DMA,
        pltpu.SemaphoreType.DMA,
    ],
    compiler_params=pltpu.CompilerParams(use_tc_tiling_on_sc=False, needs_layout_passes=False),
)
def _k2(pd_hbm, ps_hbm, src_hbm, dst_hbm, out_hbm,
        idxs0, idxs1, idxd0, idxd1, rd0, rd1, rs0, rs1,
        semi0, semi1, semg0, semg1):
    _k2_body(pd_hbm, ps_hbm, src_hbm, dst_hbm, out_hbm,
             idxs0, idxs1, idxd0, idxd1, rd0, rd1, rs0, rs1,
             semi0, semi1, semg0, semg1)


def _k3_body(zp_ref, ea_ref, w1e_ref, b1_ref, w2t_ref, htp_ref):
    pe = jnp.dot(ea_ref[...], w1e_ref[...], preferred_element_type=jnp.float32)
    a = jnp.maximum(zp_ref[...] + pe + b1_ref[...], 0.0)
    ht = lax.dot_general(
        w2t_ref[...], a, (((1,), (1,)), ((), ())),
        preferred_element_type=jnp.float32)
    hbf = ht.astype(jnp.bfloat16)
    lo = lax.bitcast_convert_type(hbf[:PAIRS, :], jnp.uint16).astype(jnp.uint32)
    hi = lax.bitcast_convert_type(hbf[PAIRS:, :], jnp.uint16).astype(jnp.uint32)
    htp_ref[...] = lax.bitcast_convert_type((hi << 16) | lo, jnp.int32)


def _k3(zp, ea, w1e, b1_2d, w2t):
    return pl.pallas_call(
        _k3_body,
        grid=(G3,),
        in_specs=[
            pl.BlockSpec((B3, OC), lambda i: (i, 0)),
            pl.BlockSpec((B3, DE), lambda i: (i, 0)),
            pl.BlockSpec((DE, OC), lambda i: (0, 0)),
            pl.BlockSpec((1, OC), lambda i: (0, 0)),
            pl.BlockSpec((OC, OC), lambda i: (0, 0)),
        ],
        out_specs=pl.BlockSpec((PAIRS, B3), lambda i: (0, i)),
        out_shape=jax.ShapeDtypeStruct((PAIRS, E), jnp.int32),
    )(zp, ea, w1e, b1_2d, w2t)


SPREAD = 8
ACC8 = SPREAD * N


def _k4_body(htp_hbm, dst_hbm, out_hbm, idx0, idx1, val0, val1, acc_v, sem0, sem1):
    wid = lax.axis_index("s") * NC + lax.axis_index("c")
    cpair = wid & (PAIRS - 1)
    ebase = (wid >> 4) * E2
    idxb = (idx0, idx1)
    valb = (val0, val1)
    sems = (sem0, sem1)
    lanes = lax.iota(jnp.int32, L)
    lane_off = (lanes & (SPREAD - 1)) * N
    neg_inf2 = jnp.full((L,), -8323200, dtype=jnp.int32)
    zeros = jnp.zeros((L,), jnp.float32)

    def bf(x):
        return plsc.bitcast(x, jnp.bfloat16)

    def ini(i, c):
        acc_v[pl.ds(i * L, L)] = neg_inf2
        return c

    lax.fori_loop(0, ACC8 // L, ini, 0)

    def issue(j, b):
        pltpu.async_copy(dst_hbm.at[pl.ds(ebase + j * CJ, CJ)], idxb[b], sems[b])
        pltpu.async_copy(htp_hbm.at[cpair, pl.ds(ebase + j * CJ, CJ)], valb[b], sems[b])

    def waitc(b):
        pltpu.make_async_copy(dst_hbm.at[pl.ds(0, CJ)], idxb[b], sems[b]).wait()
        pltpu.make_async_copy(dst_hbm.at[pl.ds(0, CJ)], valb[b], sems[b]).wait()

    def process(jj, b):
        @pl.when(jj + 1 < K4C)
        def _prefetch():
            issue(jj + 1, 1 - b)

        waitc(b)
        idx_v = idxb[b]
        val_v = valb[b]

        @plsc.parallel_loop(0, CJ // L, unroll=2)
        def _bulk(v):
            idx = idx_v[pl.ds(v * L, L)]
            val = val_v[pl.ds(v * L, L)]
            idx8 = idx + lane_off
            cur = plsc.load_gather(acc_v, [idx8])
            new = plsc.bitcast(jnp.maximum(bf(cur), bf(val)), jnp.int32)
            plsc.store_scatter(acc_v, [idx8], new)

        def wbody(go):
            @plsc.parallel_loop(0, CJ // L, unroll=2, carry=zeros)
            def bad(v, acc_bad):
                idx = idx_v[pl.ds(v * L, L)]
                val = val_v[pl.ds(v * L, L)]
                idx8 = idx + lane_off
                chk = plsc.load_gather(acc_v, [idx8])
                new = plsc.bitcast(jnp.maximum(bf(chk), bf(val)), jnp.int32)
                m = new != chk
                plsc.store_scatter(acc_v, [idx8], new, mask=m)
                return jnp.maximum(acc_bad, jnp.where(m, 1.0, 0.0))

            return jnp.max(bad) > 0.0

        lax.while_loop(lambda go: go, wbody, jnp.bool_(True))

    issue(0, 0)

    def pair(jp, c):
        process(2 * jp, 0)
        process(2 * jp + 1, 1)
        return c

    lax.fori_loop(0, K4C // 2, pair, 0)

    def merge(i, c):
        b0 = bf(acc_v[pl.ds(i * L, L)])
        for b in range(1, SPREAD):
            b0 = jnp.maximum(b0, bf(acc_v[pl.ds(b * N + i * L, L)]))
        acc_v[pl.ds(i * L, L)] = plsc.bitcast(b0, jnp.int32)
        return c

    lax.fori_loop(0, N // L, merge, 0)
    pltpu.sync_copy(acc_v.at[pl.ds(0, N)], out_hbm.at[wid])


@functools.partial(
    pl.kernel,
    mesh=plsc.VectorSubcoreMesh(core_axis_name="c", subcore_axis_name="s"),
    out_type=jax.ShapeDtypeStruct((NW, N), jnp.int32),
    scratch_types=[
        pltpu.VMEM((CJ,), jnp.int32),
        pltpu.VMEM((CJ,), jnp.int32),
        pltpu.VMEM((CJ,), jnp.int32),
        pltpu.VMEM((CJ,), jnp.int32),
        pltpu.VMEM((ACC8,), jnp.int32),
        pltpu.SemaphoreType.DMA,
        pltpu.SemaphoreType.DMA,
    ],
    compiler_params=pltpu.CompilerParams(use_tc_tiling_on_sc=False, needs_layout_passes=False),
)
def _k4(htp_hbm, dst_hbm, out_hbm, idx0, idx1, val0, val1, acc_v, sem0, sem1):
    _k4_body(htp_hbm, dst_hbm, out_hbm, idx0, idx1, val0, val1, acc_v, sem0, sem1)


def kernel(x, edge_index, edge_attr, W1, b1, W2, b2):
    src = edge_index[0]
    dst = edge_index[1]
    w_ds = jnp.concatenate([W1[:DF], W1[DF:2 * DF]], axis=1)
    w1e = W1[2 * DF:]

    pd, ps = _k1(x, w_ds)
    zp = _k2(pd, ps, src, dst)
    htp = _k3(zp, edge_attr, w1e, b1.reshape(1, OC), W2.T)
    accp = _k4(htp, dst)
    bfp = lax.bitcast_convert_type(
        lax.bitcast_convert_type(accp, jnp.uint16), jnp.bfloat16)
    m = jnp.maximum(bfp[:PAIRS], bfp[PAIRS:])
    acc = jnp.concatenate([m[:, :, 0], m[:, :, 1]], axis=0).T
    accf = acc.astype(jnp.float32)
    return jnp.where(jnp.isfinite(accf), accf + b2[None, :], 0.0)
